# baseline (speedup 1.0000x reference)
import functools

import jax
import jax.numpy as jnp
from jax import lax
from jax.experimental import pallas as pl
from jax.experimental.pallas import tpu as pltpu
from jax.experimental.pallas import tpu_sc as plsc

N = 50000
D = 592
LANES = 16
NC = 2
NS = 16
NW = NC * NS

R = 40
BPW = 40
ROWS_W = R * BPW

_SEGS = ((0, 8), (8, 12), (20, 10), (30, 7))

_mesh = plsc.VectorSubcoreMesh(core_axis_name="c", subcore_axis_name="s")


@functools.partial(
    pl.kernel,
    mesh=_mesh,
    out_type=jax.ShapeDtypeStruct((N, D), jnp.float32),
    scratch_types=[
        pltpu.VMEM((2, R, D), jnp.float32),
        pltpu.VMEM((2, R, D), jnp.float32),
        pltpu.VMEM((ROWS_W // 4 + 8, LANES), jnp.float32),
        pltpu.SemaphoreType.DMA,
        pltpu.SemaphoreType.DMA,
        pltpu.SemaphoreType.DMA,
        pltpu.SemaphoreType.DMA,
    ],
    compiler_params=pltpu.CompilerParams(use_tc_tiling_on_sc=False),
)
def _irrepwise_sc(x_hbm, w_hbm, out_hbm, xb, ob, wv, sx0, sx1, so0, so1):
    wid = lax.axis_index("s") * NC + lax.axis_index("c")
    wbase = pl.multiple_of(jnp.minimum(wid * ROWS_W, N - ROWS_W), 8)

    sx = (sx0, sx1)
    so = (so0, so1)

    wbase_al = wbase // 32 * 32
    dq = (wbase - wbase_al) // 4
    wb4 = pl.multiple_of(wbase_al // 4, 8)
    pltpu.sync_copy(w_hbm.at[pl.ds(wb4, ROWS_W // 4 + 8)], wv)

    def start_in(blk, b):
        pltpu.async_copy(x_hbm.at[pl.ds(wbase + blk * R, R)], xb.at[b], sx[b])

    def wait_in(b):
        pltpu.make_async_copy(x_hbm.at[pl.ds(wbase, R)], xb.at[b], sx[b]).wait()

    def start_out(blk, b):
        pltpu.async_copy(ob.at[b], out_hbm.at[pl.ds(wbase + blk * R, R)], so[b])

    def wait_out(b):
        pltpu.make_async_copy(ob.at[b], out_hbm.at[pl.ds(wbase, R)], so[b]).wait()

    def compute(blk, b):
        q0 = blk * (R // 4)

        @pl.loop(0, R // 4)
        def _quad(q):
            wq = wv[dq + q0 + q]
            r0 = q * 4
            for m in range(4):
                for s, (v0, nv) in enumerate(_SEGS):
                    ws = wq.at[
                        jnp.full((LANES,), m * 4 + s, jnp.int32)
                    ].get(mode="promise_in_bounds")
                    for j in range(nv):
                        c = (v0 + j) * LANES
                        ob[b, r0 + m, pl.ds(c, LANES)] = (
                            xb[b, r0 + m, pl.ds(c, LANES)] * ws
                        )

    start_in(0, 0)
    start_in(1, 1)

    wait_in(0)
    compute(0, 0)
    start_in(2, 0)
    start_out(0, 0)
    wait_in(1)
    compute(1, 1)
    start_in(3, 1)
    start_out(1, 1)

    @pl.loop(1, BPW // 2 - 1)
    def _pair(i):
        for b in (0, 1):
            blk = 2 * i + b
            wait_in(b)
            wait_out(b)
            compute(blk, b)
            start_in(blk + 2, b)
            start_out(blk, b)

    for b in (0, 1):
        blk = BPW - 2 + b
        wait_in(b)
        wait_out(b)
        compute(blk, b)
        start_out(blk, b)

    wait_out(0)
    wait_out(1)


@jax.jit
def kernel(x, w):
    wp = jnp.pad(w.reshape(N // 4, LANES), ((0, 8), (0, 0)))
    return _irrepwise_sc(x, wp)


# speedup vs baseline: 4.0725x; 4.0725x over previous
import functools

import jax
import jax.numpy as jnp
from jax import lax
from jax.experimental import pallas as pl
from jax.experimental.pallas import tpu as pltpu
from jax.experimental.pallas import tpu_sc as plsc

N = 50000
D = 592
LANES = 16
NC = 2
NS = 16
NW = NC * NS

R = 32
BPW = 50
ROWS_W = R * BPW
QPB = R // 4

_SEGS = ((0, 8), (8, 12), (20, 10), (30, 7))

_mesh = plsc.VectorSubcoreMesh(core_axis_name="c", subcore_axis_name="s")


@functools.partial(
    pl.kernel,
    mesh=_mesh,
    out_type=jax.ShapeDtypeStruct((N, D), jnp.float32),
    scratch_types=[
        pltpu.VMEM((2, R, D), jnp.float32),
        pltpu.VMEM((2, R, D), jnp.float32),
        pltpu.VMEM((2, 2 * QPB, LANES), jnp.float32),
        pltpu.SemaphoreType.DMA,
        pltpu.SemaphoreType.DMA,
        pltpu.SemaphoreType.DMA,
        pltpu.SemaphoreType.DMA,
        pltpu.SemaphoreType.DMA,
        pltpu.SemaphoreType.DMA,
    ],
)
def _irrepwise_sc(x_hbm, w_hbm, out_hbm, xb, ob, wb, sx0, sx1, sw0, sw1,
                  so0, so1):
    wid = lax.axis_index("s") * NC + lax.axis_index("c")
    wbase = pl.multiple_of(jnp.minimum(wid * ROWS_W, N - ROWS_W), 8)

    wq_base = wbase // 4
    wq_al = pl.multiple_of(wq_base // 8 * 8, 8)
    dq = wq_base - wq_al

    sx = (sx0, sx1)
    sw = (sw0, sw1)
    so = (so0, so1)

    def start_in(blk, b):
        pltpu.async_copy(x_hbm.at[pl.ds(wbase + blk * R, R)], xb.at[b], sx[b])
        pltpu.async_copy(
            w_hbm.at[pl.ds(wq_al + blk * QPB, 2 * QPB)], wb.at[b], sw[b]
        )

    def wait_in(b):
        pltpu.make_async_copy(x_hbm.at[pl.ds(wbase, R)], xb.at[b], sx[b]).wait()
        pltpu.make_async_copy(
            w_hbm.at[pl.ds(wq_al, 2 * QPB)], wb.at[b], sw[b]
        ).wait()

    def start_out(blk, b):
        pltpu.async_copy(ob.at[b], out_hbm.at[pl.ds(wbase + blk * R, R)], so[b])

    def wait_out(b):
        pltpu.make_async_copy(ob.at[b], out_hbm.at[pl.ds(wbase, R)], so[b]).wait()

    def compute(b):
        @pl.loop(0, QPB)
        def _quad(q):
            wq = wb[b, dq + q]
            r0 = q * 4
            for m in range(4):
                for s, (v0, nv) in enumerate(_SEGS):
                    ws = wq.at[
                        jnp.full((LANES,), m * 4 + s, jnp.int32)
                    ].get(mode="promise_in_bounds")
                    for j in range(nv):
                        c = (v0 + j) * LANES
                        ob[b, r0 + m, pl.ds(c, LANES)] = (
                            xb[b, r0 + m, pl.ds(c, LANES)] * ws
                        )

    start_in(0, 0)
    start_in(1, 1)

    wait_in(0)
    compute(0)
    start_in(2, 0)
    start_out(0, 0)
    wait_in(1)
    compute(1)
    start_in(3, 1)
    start_out(1, 1)

    @pl.loop(1, BPW // 2 - 1)
    def _pair(i):
        for b in (0, 1):
            blk = 2 * i + b
            wait_in(b)
            wait_out(b)
            compute(b)
            start_in(blk + 2, b)
            start_out(blk, b)

    for b in (0, 1):
        blk = BPW - 2 + b
        wait_in(b)
        wait_out(b)
        compute(b)
        start_out(blk, b)

    wait_out(0)
    wait_out(1)


@jax.jit
def kernel(x, w):
    wp = jnp.pad(w.reshape(N // 4, LANES), ((0, 8), (0, 0)))
    return _irrepwise_sc(x, wp)


# speedup vs baseline: 4.0832x; 1.0026x over previous
import functools

import jax
import jax.numpy as jnp
from jax import lax
from jax.experimental import pallas as pl
from jax.experimental.pallas import tpu as pltpu
from jax.experimental.pallas import tpu_sc as plsc

N = 50000
D = 592
LANES = 16
NC = 2
NS = 16
NW = NC * NS

R = 32
BPW = 50
ROWS_W = R * BPW
QPB = R // 4

_SEGS = ((0, 8), (8, 12), (20, 10), (30, 7))

_mesh = plsc.VectorSubcoreMesh(core_axis_name="c", subcore_axis_name="s")


@functools.partial(
    pl.kernel,
    mesh=_mesh,
    out_type=jax.ShapeDtypeStruct((N, D), jnp.float32),
    scratch_types=[
        pltpu.VMEM((2, R, D), jnp.float32),
        pltpu.VMEM((2, R, D), jnp.float32),
        pltpu.VMEM((2, 2 * QPB, LANES), jnp.float32),
        pltpu.SemaphoreType.DMA,
        pltpu.SemaphoreType.DMA,
        pltpu.SemaphoreType.DMA,
        pltpu.SemaphoreType.DMA,
        pltpu.SemaphoreType.DMA,
        pltpu.SemaphoreType.DMA,
    ],
    compiler_params=pltpu.CompilerParams(use_tc_tiling_on_sc=True),
)
def _irrepwise_sc(x_hbm, w_hbm, out_hbm, xb, ob, wb, sx0, sx1, sw0, sw1,
                  so0, so1):
    wid = lax.axis_index("s") * NC + lax.axis_index("c")
    wbase = pl.multiple_of(jnp.minimum(wid * ROWS_W, N - ROWS_W), 8)

    wq_base = wbase // 4
    wq_al = pl.multiple_of(wq_base // 8 * 8, 8)
    dq = wq_base - wq_al

    sx = (sx0, sx1)
    sw = (sw0, sw1)
    so = (so0, so1)

    def start_in(blk, b):
        pltpu.async_copy(x_hbm.at[pl.ds(wbase + blk * R, R)], xb.at[b], sx[b])
        pltpu.async_copy(
            w_hbm.at[pl.ds(wq_al + blk * QPB, 2 * QPB)], wb.at[b], sw[b]
        )

    def wait_in(b):
        pltpu.make_async_copy(x_hbm.at[pl.ds(wbase, R)], xb.at[b], sx[b]).wait()
        pltpu.make_async_copy(
            w_hbm.at[pl.ds(wq_al, 2 * QPB)], wb.at[b], sw[b]
        ).wait()

    def start_out(blk, b):
        pltpu.async_copy(ob.at[b], out_hbm.at[pl.ds(wbase + blk * R, R)], so[b])

    def wait_out(b):
        pltpu.make_async_copy(ob.at[b], out_hbm.at[pl.ds(wbase, R)], so[b]).wait()

    def compute(b):
        @pl.loop(0, QPB)
        def _quad(q):
            wq = wb[b, dq + q]
            r0 = q * 4
            for m in range(4):
                for s, (v0, nv) in enumerate(_SEGS):
                    ws = wq.at[
                        jnp.full((LANES,), m * 4 + s, jnp.int32)
                    ].get(mode="promise_in_bounds")
                    for j in range(nv):
                        c = (v0 + j) * LANES
                        ob[b, r0 + m, pl.ds(c, LANES)] = (
                            xb[b, r0 + m, pl.ds(c, LANES)] * ws
                        )

    start_in(0, 0)
    start_in(1, 1)

    wait_in(0)
    compute(0)
    start_in(2, 0)
    start_out(0, 0)
    wait_in(1)
    compute(1)
    start_in(3, 1)
    start_out(1, 1)

    @pl.loop(1, BPW // 2 - 1)
    def _pair(i):
        for b in (0, 1):
            blk = 2 * i + b
            wait_in(b)
            wait_out(b)
            compute(b)
            start_in(blk + 2, b)
            start_out(blk, b)

    for b in (0, 1):
        blk = BPW - 2 + b
        wait_in(b)
        wait_out(b)
        compute(b)
        start_out(blk, b)

    wait_out(0)
    wait_out(1)


@jax.jit
def kernel(x, w):
    wp = jnp.pad(w.reshape(N // 4, LANES), ((0, 8), (0, 0)))
    return _irrepwise_sc(x, wp)


# speedup vs baseline: 11.8907x; 2.9121x over previous
import functools

import jax
import jax.numpy as jnp
from jax import lax
from jax.experimental import pallas as pl
from jax.experimental.pallas import tpu as pltpu
from jax.experimental.pallas import tpu_sc as plsc

N = 50000
D = 592
LANES = 16
NC = 2
NS = 16
NW = NC * NS

CB = 1664
NTAIL = 80
TAIL0 = N - NTAIL
CLAMP = TAIL0 - CB
UNITS = D // 8

_mesh = plsc.VectorSubcoreMesh(core_axis_name="c", subcore_axis_name="s")


def _seg_of_unit(u):
    if isinstance(u, int):
        return int(u >= 16) + int(u >= 40) + int(u >= 60)
    return (
        (u >= 16).astype(jnp.int32)
        + (u >= 40).astype(jnp.int32)
        + (u >= 60).astype(jnp.int32)
    )


@functools.partial(
    pl.kernel,
    mesh=_mesh,
    out_type=jax.ShapeDtypeStruct((D, N), jnp.float32),
    scratch_types=[
        pltpu.VMEM((2, 8, CB), jnp.float32),
        pltpu.VMEM((2, 8, CB), jnp.float32),
        pltpu.VMEM((4, CB), jnp.float32),
        pltpu.VMEM((D // 2, NTAIL), jnp.float32),
        pltpu.VMEM((4, NTAIL), jnp.float32),
        pltpu.SemaphoreType.DMA,
        pltpu.SemaphoreType.DMA,
        pltpu.SemaphoreType.DMA,
        pltpu.SemaphoreType.DMA,
    ],
    compiler_params=pltpu.CompilerParams(use_tc_tiling_on_sc=True),
)
def _irrepwise_sc_t(xt_hbm, wt_hbm, out_hbm, xb, ob, wv, tb, twv,
                    sx0, sx1, so0, so1):
    wid = lax.axis_index("s") * NC + lax.axis_index("c")
    cbase = pl.multiple_of(jnp.minimum(wid * CB, CLAMP), 128)

    sx = (sx0, sx1)
    so = (so0, so1)

    pltpu.sync_copy(wt_hbm.at[:, pl.ds(cbase, CB)], wv)

    def start_in(u, b):
        j0 = pl.multiple_of(u * 8, 8)
        pltpu.async_copy(
            xt_hbm.at[pl.ds(j0, 8), pl.ds(cbase, CB)], xb.at[b], sx[b]
        )

    def wait_in(b):
        pltpu.make_async_copy(
            xt_hbm.at[pl.ds(0, 8), pl.ds(cbase, CB)], xb.at[b], sx[b]
        ).wait()

    def start_out(u, b):
        j0 = pl.multiple_of(u * 8, 8)
        pltpu.async_copy(
            ob.at[b], out_hbm.at[pl.ds(j0, 8), pl.ds(cbase, CB)], so[b]
        )

    def wait_out(b):
        pltpu.make_async_copy(
            ob.at[b], out_hbm.at[pl.ds(0, 8), pl.ds(cbase, CB)], so[b]
        ).wait()

    def compute(u, b):
        s = _seg_of_unit(u)

        @pl.loop(0, CB // LANES)
        def _chunk(k):
            c = k * LANES
            wk = wv[s, pl.ds(c, LANES)]
            for r in range(8):
                ob[b, r, pl.ds(c, LANES)] = xb[b, r, pl.ds(c, LANES)] * wk

    start_in(0, 0)
    start_in(1, 1)

    wait_in(0)
    compute(0, 0)
    start_in(2, 0)
    start_out(0, 0)
    wait_in(1)
    compute(1, 1)
    start_in(3, 1)
    start_out(1, 1)

    @pl.loop(1, UNITS // 2 - 1)
    def _pair(i):
        for b in (0, 1):
            u = 2 * i + b
            wait_in(b)
            wait_out(b)
            compute(u, b)
            start_in(u + 2, b)
            start_out(u, b)

    for b in (0, 1):
        u = UNITS - 2 + b
        wait_in(b)
        wait_out(b)
        compute(u, b)
        start_out(u, b)

    wait_out(0)
    wait_out(1)

    @pl.when(wid == 0)
    def _tail():
        pltpu.sync_copy(wt_hbm.at[:, pl.ds(TAIL0, NTAIL)], twv)
        for h in (0, 1):
            row0 = h * (D // 2)
            pltpu.sync_copy(
                xt_hbm.at[pl.ds(row0, D // 2), pl.ds(TAIL0, NTAIL)], tb
            )

            @pl.loop(0, UNITS // 2)
            def _u(u):
                s = _seg_of_unit(row0 // 8 + u)
                j0 = u * 8
                for k in range(NTAIL // LANES):
                    c = k * LANES
                    wk = twv[s, pl.ds(c, LANES)]
                    for r in range(8):
                        tb[j0 + r, pl.ds(c, LANES)] = (
                            tb[j0 + r, pl.ds(c, LANES)] * wk
                        )

            pltpu.sync_copy(
                tb, out_hbm.at[pl.ds(row0, D // 2), pl.ds(TAIL0, NTAIL)]
            )


@jax.jit
def kernel(x, w):
    return _irrepwise_sc_t(x.T, w.T).T


# speedup vs baseline: 13.1452x; 1.1055x over previous
import functools

import jax
import jax.numpy as jnp
from jax import lax
from jax.experimental import pallas as pl
from jax.experimental.pallas import tpu as pltpu
from jax.experimental.pallas import tpu_sc as plsc

N = 50000
D = 592
LANES = 16
NC = 2
NS = 16
NW = NC * NS

CB = 1664
NTAIL = 80
TAIL0 = N - NTAIL
UNITS = D // 8
TROWS = D // 2

_mesh = plsc.VectorSubcoreMesh(core_axis_name="c", subcore_axis_name="s")


def _seg_of_unit(u):
    if isinstance(u, int):
        return int(u >= 16) + int(u >= 40) + int(u >= 60)
    return (
        (u >= 16).astype(jnp.int32)
        + (u >= 40).astype(jnp.int32)
        + (u >= 60).astype(jnp.int32)
    )


@functools.partial(
    pl.kernel,
    mesh=_mesh,
    out_type=jax.ShapeDtypeStruct((D, N), jnp.float32),
    scratch_types=[
        pltpu.VMEM((2, 8, CB), jnp.float32),
        pltpu.VMEM((2, 8, CB), jnp.float32),
        pltpu.VMEM((4, CB), jnp.float32),
        pltpu.VMEM((TROWS, NTAIL), jnp.float32),
        pltpu.VMEM((4, NTAIL), jnp.float32),
        pltpu.SemaphoreType.DMA,
        pltpu.SemaphoreType.DMA,
        pltpu.SemaphoreType.DMA,
        pltpu.SemaphoreType.DMA,
    ],
    compiler_params=pltpu.CompilerParams(use_tc_tiling_on_sc=True),
)
def _irrepwise_sc_t(xt_hbm, wt_hbm, out_hbm, xb, ob, wv, tb, twv,
                    sx0, sx1, so0, so1):
    wid = lax.axis_index("s") * NC + lax.axis_index("c")
    cbase = pl.multiple_of(jnp.minimum(wid, 29) * CB, 128)

    sx = (sx0, sx1)
    so = (so0, so1)

    def start_in(u, b):
        j0 = pl.multiple_of(u * 8, 8)
        pltpu.async_copy(
            xt_hbm.at[pl.ds(j0, 8), pl.ds(cbase, CB)], xb.at[b], sx[b]
        )

    def wait_in(b):
        pltpu.make_async_copy(
            xt_hbm.at[pl.ds(0, 8), pl.ds(cbase, CB)], xb.at[b], sx[b]
        ).wait()

    def start_out(u, b):
        j0 = pl.multiple_of(u * 8, 8)
        pltpu.async_copy(
            ob.at[b], out_hbm.at[pl.ds(j0, 8), pl.ds(cbase, CB)], so[b]
        )

    def wait_out(b):
        pltpu.make_async_copy(
            ob.at[b], out_hbm.at[pl.ds(0, 8), pl.ds(cbase, CB)], so[b]
        ).wait()

    def compute(u, b):
        s = _seg_of_unit(u)

        @pl.loop(0, CB // LANES)
        def _chunk(k):
            c = k * LANES
            wk = wv[s, pl.ds(c, LANES)]
            for r in range(8):
                ob[b, r, pl.ds(c, LANES)] = xb[b, r, pl.ds(c, LANES)] * wk

    @pl.when(wid < 30)
    def _main():
        pltpu.sync_copy(wt_hbm.at[:, pl.ds(cbase, CB)], wv)

        start_in(0, 0)
        start_in(1, 1)

        wait_in(0)
        compute(0, 0)
        start_in(2, 0)
        start_out(0, 0)
        wait_in(1)
        compute(1, 1)
        start_in(3, 1)
        start_out(1, 1)

        @pl.loop(1, UNITS // 2 - 1)
        def _pair(i):
            for b in (0, 1):
                u = 2 * i + b
                wait_in(b)
                wait_out(b)
                compute(u, b)
                start_in(u + 2, b)
                start_out(u, b)

        for b in (0, 1):
            u = UNITS - 2 + b
            wait_in(b)
            wait_out(b)
            compute(u, b)
            start_out(u, b)

        wait_out(0)
        wait_out(1)

    @pl.when(wid >= 30)
    def _tail():
        row0 = pl.multiple_of((wid - 30) * TROWS, 8)
        u0 = (wid - 30) * (TROWS // 8)
        pltpu.sync_copy(wt_hbm.at[:, pl.ds(TAIL0, NTAIL)], twv)
        pltpu.sync_copy(
            xt_hbm.at[pl.ds(row0, TROWS), pl.ds(TAIL0, NTAIL)], tb
        )

        @pl.loop(0, TROWS // 8)
        def _u(u):
            s = _seg_of_unit(u0 + u)
            j0 = u * 8
            for k in range(NTAIL // LANES):
                c = k * LANES
                wk = twv[s, pl.ds(c, LANES)]
                for r in range(8):
                    tb[j0 + r, pl.ds(c, LANES)] = (
                        tb[j0 + r, pl.ds(c, LANES)] * wk
                    )

        pltpu.sync_copy(
            tb, out_hbm.at[pl.ds(row0, TROWS), pl.ds(TAIL0, NTAIL)]
        )


@jax.jit
def kernel(x, w):
    return _irrepwise_sc_t(x.T, w.T).T
